# Initial kernel scaffold; baseline (speedup 1.0000x reference)
import functools

import jax
import jax.numpy as jnp
from jax import lax
from jax.experimental import pallas as pl
from jax.experimental.pallas import tpu as pltpu
from jax.experimental.pallas import tpu_sc as plsc

N_NODES = 10000
N_EDGES = 320000
V_IN = 128
D = 128
EF = 16
H = 64
NC = 2
NS = 16
EB = 80
EDGES_PER_TILE = N_EDGES // NS
NBATCH = EDGES_PER_TILE // EB
NODES_PER_TILE = N_NODES // NS
EPI_CHUNK = 125
NODE_BLK = 400
EDGE_BLK = 3200


def _node_tables_body(x_ref, ws_ref, wd_ref, we_ref, src_ref, dst_ref, xe_ref):
    xb = x_ref[...]
    src_ref[0] = jnp.dot(xb, ws_ref[0], preferred_element_type=jnp.float32)
    src_ref[1] = jnp.dot(xb, ws_ref[1], preferred_element_type=jnp.float32)
    dst_ref[0] = jnp.dot(xb, wd_ref[0], preferred_element_type=jnp.float32)
    dst_ref[1] = jnp.dot(xb, wd_ref[1], preferred_element_type=jnp.float32)
    xe_ref[...] = jnp.dot(xb, we_ref[...], preferred_element_type=jnp.float32)


def _node_tables(x, ws, wd, we):
    nblk = N_NODES // NODE_BLK
    return pl.pallas_call(
        _node_tables_body,
        grid=(nblk,),
        in_specs=[
            pl.BlockSpec((NODE_BLK, V_IN), lambda i: (i, 0)),
            pl.BlockSpec((NC, V_IN, D), lambda i: (0, 0, 0)),
            pl.BlockSpec((NC, V_IN, D), lambda i: (0, 0, 0)),
            pl.BlockSpec((V_IN, EF), lambda i: (0, 0)),
        ],
        out_specs=[
            pl.BlockSpec((NC, NODE_BLK, D), lambda i: (0, i, 0)),
            pl.BlockSpec((NC, NODE_BLK, D), lambda i: (0, i, 0)),
            pl.BlockSpec((NODE_BLK, EF), lambda i: (i, 0)),
        ],
        out_shape=[
            jax.ShapeDtypeStruct((NC, N_NODES, D), jnp.float32),
            jax.ShapeDtypeStruct((NC, N_NODES, D), jnp.float32),
            jax.ShapeDtypeStruct((N_NODES, EF), jnp.float32),
        ],
    )(x, ws, wd, we)


def _edge_tables_body(e_ref, wa_ref, wee_ref, edg_ref, ee_ref, mx_ref, mn_ref):
    eb = e_ref[...]
    o0 = jnp.dot(eb, wa_ref[0], preferred_element_type=jnp.float32)
    o1 = jnp.dot(eb, wa_ref[1], preferred_element_type=jnp.float32)
    edg_ref[0] = o0
    edg_ref[1] = o1
    ee_ref[...] = jnp.dot(eb, wee_ref[...], preferred_element_type=jnp.float32)
    acat = jnp.concatenate([o0[:, :H], o1[:, :H]], axis=1)
    mx_ref[0] = jnp.broadcast_to(jnp.max(acat, axis=0, keepdims=True), (8, D))
    mn_ref[0] = jnp.broadcast_to(jnp.min(acat, axis=0, keepdims=True), (8, D))


def _edge_tables(e, wa, wee):
    nblk = N_EDGES // EDGE_BLK
    return pl.pallas_call(
        _edge_tables_body,
        grid=(nblk,),
        in_specs=[
            pl.BlockSpec((EDGE_BLK, EF), lambda i: (i, 0)),
            pl.BlockSpec((NC, EF, D), lambda i: (0, 0, 0)),
            pl.BlockSpec((EF, EF), lambda i: (0, 0)),
        ],
        out_specs=[
            pl.BlockSpec((NC, EDGE_BLK, D), lambda i: (0, i, 0)),
            pl.BlockSpec((EDGE_BLK, EF), lambda i: (i, 0)),
            pl.BlockSpec((1, 8, D), lambda i: (i, 0, 0)),
            pl.BlockSpec((1, 8, D), lambda i: (i, 0, 0)),
        ],
        out_shape=[
            jax.ShapeDtypeStruct((NC, N_EDGES, D), jnp.float32),
            jax.ShapeDtypeStruct((N_EDGES, EF), jnp.float32),
            jax.ShapeDtypeStruct((nblk, 8, D), jnp.float32),
            jax.ShapeDtypeStruct((nblk, 8, D), jnp.float32),
        ],
    )(e, wa, wee)


def _sc_body(src_tab, dst_tab, edg_tab, xe_tab, ee_tab, src_idx, dst_idx,
             m_hbm, bt_hbm, pw_hbm,
             out_x, out_e,
             s0_acc, s1_acc,
             srcv, srcva, dstv, dstva,
             srcrows, dstrows, edgrows, scat0, scat1,
             xsrows, xdrows, eerows, nerows,
             mvec, btvec, pwvec, eb0, eb1,
             sem1, sem2):
    ci = lax.axis_index("c")
    si = lax.axis_index("s")
    mbase = ci * H

    pltpu.sync_copy(m_hbm, mvec)
    pltpu.sync_copy(bt_hbm, btvec)
    pltpu.sync_copy(pw_hbm, pwvec)
    pwv = pwvec[...]
    zero16 = jnp.zeros((16,), jnp.float32)

    def _zbody(i, _):
        r = lax.shift_right_logical(i, 2)
        co = jnp.bitwise_and(i, 3) * 16
        eb0[r, pl.ds(co, 16)] = zero16
        return 0
    lax.fori_loop(0, EPI_CHUNK * 4, _zbody, 0)
    for k in range(NODES_PER_TILE // EPI_CHUNK):
        base = si * NODES_PER_TILE + k * EPI_CHUNK
        pltpu.sync_copy(eb0, s0_acc.at[pl.ds(base, EPI_CHUNK)])
        pltpu.sync_copy(eb0, s1_acc.at[pl.ds(base, EPI_CHUNK)])
    plsc.subcore_barrier()

    ebase = si * EDGES_PER_TILE
    node_off = ci * N_NODES
    edge_off = ci * N_EDGES

    def _batch(nb, _):
        start = ebase + nb * EB
        pltpu.sync_copy(src_idx.at[pl.ds(start, EB)], srcv)
        pltpu.sync_copy(dst_idx.at[pl.ds(start, EB)], dstv)

        offv = jnp.full((16,), node_off, jnp.int32)

        def _adj(k, _):
            co = k * 16
            srcva[pl.ds(co, 16)] = srcv[pl.ds(co, 16)] + offv
            dstva[pl.ds(co, 16)] = dstv[pl.ds(co, 16)] + offv
            return 0
        lax.fori_loop(0, EB // 16, _adj, 0)

        cp1 = pltpu.async_copy(src_tab.at[srcva], srcrows, sem1)
        cp2 = pltpu.async_copy(dst_tab.at[dstva], dstrows, sem2)
        pltpu.sync_copy(edg_tab.at[pl.ds(edge_off + start, EB)], edgrows)
        cp1.wait()
        cp2.wait()

        def _cbody(i, _):
            b = lax.shift_right_logical(i, 2)
            co = jnp.bitwise_and(i, 3) * 16
            a1 = srcrows[b, pl.ds(co, 16)]
            a2 = dstrows[b, pl.ds(co, 16)]
            ae = edgrows[b, pl.ds(co, 16)]
            lin = a1 + a2 + ae
            logit = jnp.where(lin >= 0.0, lin, pwv * lin)
            mv = mvec[pl.ds(mbase + co, 16)]
            ex = jnp.exp(logit - mv)
            t1 = srcrows[b, pl.ds(co + H, 16)]
            t2 = dstrows[b, pl.ds(co + H, 16)]
            te = edgrows[b, pl.ds(co + H, 16)]
            scat0[b, pl.ds(co, 16)] = ex
            scat1[b, pl.ds(co, 16)] = ex * (t1 + t2 + te)
            return 0
        lax.fori_loop(0, EB * 4, _cbody, 0)

        pltpu.sync_copy(scat0, s0_acc.at[dstv], add=True)
        pltpu.sync_copy(scat1, s1_acc.at[dstv], add=True)

        do_ne = jnp.logical_xor(ci == 1, nb < (NBATCH // 2))

        @pl.when(do_ne)
        def _():
            cp3 = pltpu.async_copy(xe_tab.at[srcv], xsrows, sem1)
            cp4 = pltpu.async_copy(xe_tab.at[dstv], xdrows, sem2)
            pltpu.sync_copy(ee_tab.at[pl.ds(start, EB)], eerows)
            cp3.wait()
            cp4.wait()

            def _nbody(b, _):
                nerows[b, :] = xsrows[b, :] + xdrows[b, :] + eerows[b, :]
                return 0
            lax.fori_loop(0, EB, _nbody, 0)
            pltpu.sync_copy(nerows, out_e.at[pl.ds(start, EB)])
        return 0

    lax.fori_loop(0, NBATCH, _batch, 0)
    plsc.subcore_barrier()

    eps = jnp.full((16,), 1e-16, jnp.float32)
    for k in range(NODES_PER_TILE // EPI_CHUNK):
        base = si * NODES_PER_TILE + k * EPI_CHUNK
        pltpu.sync_copy(s0_acc.at[pl.ds(base, EPI_CHUNK)], eb0)
        pltpu.sync_copy(s1_acc.at[pl.ds(base, EPI_CHUNK)], eb1)

        def _ebody(i, _):
            r = lax.shift_right_logical(i, 2)
            co = jnp.bitwise_and(i, 3) * 16
            s0 = eb0[r, pl.ds(co, 16)]
            s1 = eb1[r, pl.ds(co, 16)]
            bt = btvec[pl.ds(mbase + co, 16)]
            eb1[r, pl.ds(co, 16)] = s1 / (s0 + eps) + bt
            return 0
        lax.fori_loop(0, EPI_CHUNK * 4, _ebody, 0)
        pltpu.sync_copy(eb1, out_x.at[pl.ds(node_off + base, EPI_CHUNK)])


_sc_pass = functools.partial(
    pl.kernel,
    out_type=[
        jax.ShapeDtypeStruct((NC * N_NODES, H), jnp.float32),
        jax.ShapeDtypeStruct((N_EDGES, EF), jnp.float32),
    ],
    mesh=plsc.VectorSubcoreMesh(
        core_axis_name="c", subcore_axis_name="s", num_cores=NC,
        num_subcores=NS),
    scratch_types=[
        pltpu.VMEM_SHARED((N_NODES, H), jnp.float32),
        pltpu.VMEM_SHARED((N_NODES, H), jnp.float32),
        pltpu.VMEM((EB,), jnp.int32),
        pltpu.VMEM((EB,), jnp.int32),
        pltpu.VMEM((EB,), jnp.int32),
        pltpu.VMEM((EB,), jnp.int32),
        pltpu.VMEM((EB, D), jnp.float32),
        pltpu.VMEM((EB, D), jnp.float32),
        pltpu.VMEM((EB, D), jnp.float32),
        pltpu.VMEM((EB, H), jnp.float32),
        pltpu.VMEM((EB, H), jnp.float32),
        pltpu.VMEM((EB, EF), jnp.float32),
        pltpu.VMEM((EB, EF), jnp.float32),
        pltpu.VMEM((EB, EF), jnp.float32),
        pltpu.VMEM((EB, EF), jnp.float32),
        pltpu.VMEM((D,), jnp.float32),
        pltpu.VMEM((D,), jnp.float32),
        pltpu.VMEM((16,), jnp.float32),
        pltpu.VMEM((EPI_CHUNK, H), jnp.float32),
        pltpu.VMEM((EPI_CHUNK, H), jnp.float32),
        pltpu.SemaphoreType.DMA,
        pltpu.SemaphoreType.DMA,
    ],
)(_sc_body)


def kernel(x, edge_index, edge_attr, W_a, W_T, b_T, W_e, W_ee, prelu_w):
    x = x.astype(jnp.float32)
    e = edge_attr.astype(jnp.float32)
    src = edge_index[0].astype(jnp.int32)
    dst = edge_index[1].astype(jnp.int32)

    A2, Ae, A1 = W_a[:V_IN], W_a[V_IN:V_IN + EF], W_a[V_IN + EF:]
    T2, Te, T1 = W_T[:V_IN], W_T[V_IN:V_IN + EF], W_T[V_IN + EF:]

    def halves(a_part, t_part):
        return jnp.stack([
            jnp.concatenate([a_part[:, :H], t_part[:, :H]], axis=1),
            jnp.concatenate([a_part[:, H:], t_part[:, H:]], axis=1),
        ])

    ws = halves(A1, T1)
    wd = halves(A2, T2)
    wa = halves(Ae, Te)

    src_pair, dst_pair, xe = _node_tables(x, ws, wd, W_e)
    edg_pair, ee, amx, amn = _edge_tables(e, wa, W_ee)

    smax = jnp.concatenate([src_pair[0, :, :H].max(0), src_pair[1, :, :H].max(0)])
    smin = jnp.concatenate([src_pair[0, :, :H].min(0), src_pair[1, :, :H].min(0)])
    dmax = jnp.concatenate([dst_pair[0, :, :H].max(0), dst_pair[1, :, :H].max(0)])
    dmin = jnp.concatenate([dst_pair[0, :, :H].min(0), dst_pair[1, :, :H].min(0)])
    emax = amx.max(axis=(0, 1))
    emin = amn.min(axis=(0, 1))
    hi = smax + dmax + emax
    lo = smin + dmin + emin
    mvec = jnp.maximum(hi, jnp.maximum(prelu_w * hi, prelu_w * lo))
    mvec = mvec.astype(jnp.float32)

    src_tab = src_pair.reshape(NC * N_NODES, D)
    dst_tab = dst_pair.reshape(NC * N_NODES, D)
    edg_tab = edg_pair.reshape(NC * N_EDGES, D)
    pwv = jnp.full((16,), prelu_w, jnp.float32)

    out_x, out_e = _sc_pass(src_tab, dst_tab, edg_tab, xe, ee, src, dst,
                            mvec, b_T.astype(jnp.float32), pwv)

    new_x = jnp.concatenate([out_x[:N_NODES], out_x[N_NODES:]], axis=1)
    return (new_x, out_e)


# speedup vs baseline: 1.7276x; 1.7276x over previous
import functools

import jax
import jax.numpy as jnp
from jax import lax
from jax.experimental import pallas as pl
from jax.experimental.pallas import tpu as pltpu
from jax.experimental.pallas import tpu_sc as plsc

N_NODES = 10000
N_EDGES = 320000
V_IN = 128
D = 128
EF = 16
H = 64
NC = 2
NS = 16
EB = 40
EDGES_PER_TILE = N_EDGES // NS
NBATCH = EDGES_PER_TILE // EB
NPAD = 10240
NODES_PER_TILE = NPAD // NS
EPI_CHUNK = 64
NODE_BLK = 400
EDGE_BLK = 3200


def _node_tables_body(x_ref, ws_ref, wd_ref, we_ref, src_ref, dst_ref, xe_ref):
    xb = x_ref[...]
    src_ref[0] = jnp.dot(xb, ws_ref[0], preferred_element_type=jnp.float32)
    src_ref[1] = jnp.dot(xb, ws_ref[1], preferred_element_type=jnp.float32)
    dst_ref[0] = jnp.dot(xb, wd_ref[0], preferred_element_type=jnp.float32)
    dst_ref[1] = jnp.dot(xb, wd_ref[1], preferred_element_type=jnp.float32)
    xe_ref[...] = jnp.dot(xb, we_ref[...], preferred_element_type=jnp.float32)


def _node_tables(x, ws, wd, we):
    nblk = N_NODES // NODE_BLK
    return pl.pallas_call(
        _node_tables_body,
        grid=(nblk,),
        in_specs=[
            pl.BlockSpec((NODE_BLK, V_IN), lambda i: (i, 0)),
            pl.BlockSpec((NC, V_IN, D), lambda i: (0, 0, 0)),
            pl.BlockSpec((NC, V_IN, D), lambda i: (0, 0, 0)),
            pl.BlockSpec((V_IN, D), lambda i: (0, 0)),
        ],
        out_specs=[
            pl.BlockSpec((NC, NODE_BLK, D), lambda i: (0, i, 0)),
            pl.BlockSpec((NC, NODE_BLK, D), lambda i: (0, i, 0)),
            pl.BlockSpec((NODE_BLK, D), lambda i: (i, 0)),
        ],
        out_shape=[
            jax.ShapeDtypeStruct((NC, N_NODES, D), jnp.float32),
            jax.ShapeDtypeStruct((NC, N_NODES, D), jnp.float32),
            jax.ShapeDtypeStruct((N_NODES, D), jnp.float32),
        ],
    )(x, ws, wd, we)


def _edge_tables_body(e_ref, wa_ref, wee_ref, edg_ref, ee_ref, mx_ref, mn_ref):
    eb = e_ref[...]
    o0 = jnp.dot(eb, wa_ref[0], preferred_element_type=jnp.float32)
    o1 = jnp.dot(eb, wa_ref[1], preferred_element_type=jnp.float32)
    edg_ref[0] = o0
    edg_ref[1] = o1
    ee_ref[...] = jnp.dot(eb, wee_ref[...], preferred_element_type=jnp.float32)
    acat = jnp.concatenate([o0[:, :H], o1[:, :H]], axis=1)
    mx_ref[0] = jnp.broadcast_to(jnp.max(acat, axis=0, keepdims=True), (8, D))
    mn_ref[0] = jnp.broadcast_to(jnp.min(acat, axis=0, keepdims=True), (8, D))


def _edge_tables(e, wa, wee):
    nblk = N_EDGES // EDGE_BLK
    return pl.pallas_call(
        _edge_tables_body,
        grid=(nblk,),
        in_specs=[
            pl.BlockSpec((EDGE_BLK, EF), lambda i: (i, 0)),
            pl.BlockSpec((NC, EF, D), lambda i: (0, 0, 0)),
            pl.BlockSpec((EF, EF), lambda i: (0, 0)),
        ],
        out_specs=[
            pl.BlockSpec((NC, EDGE_BLK, D), lambda i: (0, i, 0)),
            pl.BlockSpec((EDGE_BLK, EF), lambda i: (i, 0)),
            pl.BlockSpec((1, 8, D), lambda i: (i, 0, 0)),
            pl.BlockSpec((1, 8, D), lambda i: (i, 0, 0)),
        ],
        out_shape=[
            jax.ShapeDtypeStruct((NC, N_EDGES, D), jnp.float32),
            jax.ShapeDtypeStruct((N_EDGES, EF), jnp.float32),
            jax.ShapeDtypeStruct((nblk, 8, D), jnp.float32),
            jax.ShapeDtypeStruct((nblk, 8, D), jnp.float32),
        ],
    )(e, wa, wee)


def _sc_body(src_tab, dst_tab, edg_tab, xe_tab, ee_tab,
             srcg_idx, dstg_idx, dst_idx,
             m_hbm, bt_hbm, pw_hbm,
             out_x, out_e,
             s_acc,
             srcv, dstva, dstv,
             srcrows, dstrows, edgrows, scat,
             eerows, ebo,
             mvec, btvec, pwvec,
             sem1, sem2):
    ci = lax.axis_index("c")
    si = lax.axis_index("s")
    mbase = ci * H

    pltpu.sync_copy(m_hbm, mvec)
    pltpu.sync_copy(bt_hbm, btvec)
    pltpu.sync_copy(pw_hbm, pwvec)
    pwv = pwvec[...]
    zero16 = jnp.zeros((16,), jnp.float32)

    @pl.loop(0, EB * 8)
    def _zbody(i):
        r = lax.shift_right_logical(i, 3)
        co = jnp.bitwise_and(i, 7) * 16
        scat[r, pl.ds(co, 16)] = zero16

    for k in range(NODES_PER_TILE // EB):
        base = si * NODES_PER_TILE + k * EB
        pltpu.sync_copy(scat, s_acc.at[pl.ds(base, EB)])
    plsc.subcore_barrier()

    ebase = si * EDGES_PER_TILE
    idx_off = ci * N_EDGES

    @pl.loop(0, NBATCH)
    def _batch(nb):
        start = ebase + nb * EB
        pltpu.sync_copy(srcg_idx.at[pl.ds(idx_off + start, EB)], srcv)
        pltpu.sync_copy(dstg_idx.at[pl.ds(idx_off + start, EB)], dstva)
        pltpu.sync_copy(dst_idx.at[pl.ds(start, EB)], dstv)

        cp1 = pltpu.async_copy(src_tab.at[srcv], srcrows, sem1)
        cp2 = pltpu.async_copy(dst_tab.at[dstva], dstrows, sem2)
        pltpu.sync_copy(edg_tab.at[pl.ds(idx_off + start, EB)], edgrows)
        cp1.wait()
        cp2.wait()

        @pl.loop(0, EB * 4)
        def _cbody(i):
            b = lax.shift_right_logical(i, 2)
            co = jnp.bitwise_and(i, 3) * 16
            a1 = srcrows[b, pl.ds(co, 16)]
            a2 = dstrows[b, pl.ds(co, 16)]
            ae = edgrows[b, pl.ds(co, 16)]
            lin = a1 + a2 + ae
            logit = jnp.where(lin >= 0.0, lin, pwv * lin)
            mv = mvec[pl.ds(mbase + co, 16)]
            ex = jnp.exp(logit - mv)
            t1 = srcrows[b, pl.ds(co + H, 16)]
            t2 = dstrows[b, pl.ds(co + H, 16)]
            te = edgrows[b, pl.ds(co + H, 16)]
            scat[b, pl.ds(co, 16)] = ex
            scat[b, pl.ds(co + H, 16)] = ex * (t1 + t2 + te)

        pltpu.sync_copy(scat, s_acc.at[dstv], add=True)

    wid = si * NC + ci
    nbase = wid * (N_EDGES // (NC * NS))

    @pl.loop(0, N_EDGES // (NC * NS * EB))
    def _nebatch(nb):
        start = nbase + nb * EB
        pltpu.sync_copy(dst_idx.at[pl.ds(start, EB)], dstv)
        pltpu.sync_copy(srcg_idx.at[pl.ds(start, EB)], srcv)
        cp3 = pltpu.async_copy(xe_tab.at[srcv], srcrows, sem1)
        cp4 = pltpu.async_copy(xe_tab.at[dstv], dstrows, sem2)
        pltpu.sync_copy(ee_tab.at[pl.ds(start, EB)], eerows)
        cp3.wait()
        cp4.wait()

        @pl.loop(0, EB)
        def _nbody(b):
            eerows[b, :] = (srcrows[b, pl.ds(0, EF)] +
                            dstrows[b, pl.ds(0, EF)] + eerows[b, :])

        pltpu.sync_copy(eerows, out_e.at[pl.ds(start, EB)])

    plsc.subcore_barrier()

    eps = jnp.full((16,), 1e-16, jnp.float32)
    for k in range(NODES_PER_TILE // EB):
        base = si * NODES_PER_TILE + k * EB
        pltpu.sync_copy(s_acc.at[pl.ds(base, EB)], srcrows)

        @pl.loop(0, EB * 4)
        def _ebody(i):
            r = lax.shift_right_logical(i, 2)
            co = jnp.bitwise_and(i, 3) * 16
            s0 = srcrows[r, pl.ds(co, 16)]
            s1 = srcrows[r, pl.ds(co + H, 16)]
            bt = btvec[pl.ds(mbase + co, 16)]
            ebo[r, pl.ds(co, 16)] = s1 / (s0 + eps) + bt

        pltpu.sync_copy(ebo, out_x.at[pl.ds(ci * NPAD + base, EB)])


_sc_pass = functools.partial(
    pl.kernel,
    out_type=[
        jax.ShapeDtypeStruct((NC * NPAD, H), jnp.float32),
        jax.ShapeDtypeStruct((N_EDGES, EF), jnp.float32),
    ],
    mesh=plsc.VectorSubcoreMesh(
        core_axis_name="c", subcore_axis_name="s", num_cores=NC,
        num_subcores=NS),
    scratch_types=[
        pltpu.VMEM_SHARED((NPAD, D), jnp.float32),
        pltpu.VMEM((EB,), jnp.int32),
        pltpu.VMEM((EB,), jnp.int32),
        pltpu.VMEM((EB,), jnp.int32),
        pltpu.VMEM((EB, D), jnp.float32),
        pltpu.VMEM((EB, D), jnp.float32),
        pltpu.VMEM((EB, D), jnp.float32),
        pltpu.VMEM((EB, D), jnp.float32),
        pltpu.VMEM((EB, EF), jnp.float32),
        pltpu.VMEM((EB, H), jnp.float32),
        pltpu.VMEM((D,), jnp.float32),
        pltpu.VMEM((D,), jnp.float32),
        pltpu.VMEM((16,), jnp.float32),
        pltpu.SemaphoreType.DMA,
        pltpu.SemaphoreType.DMA,
    ],
)(_sc_body)


def kernel(x, edge_index, edge_attr, W_a, W_T, b_T, W_e, W_ee, prelu_w):
    x = x.astype(jnp.float32)
    e = edge_attr.astype(jnp.float32)
    src = edge_index[0].astype(jnp.int32)
    dst = edge_index[1].astype(jnp.int32)

    A2, Ae, A1 = W_a[:V_IN], W_a[V_IN:V_IN + EF], W_a[V_IN + EF:]
    T2, Te, T1 = W_T[:V_IN], W_T[V_IN:V_IN + EF], W_T[V_IN + EF:]

    def halves(a_part, t_part):
        return jnp.stack([
            jnp.concatenate([a_part[:, :H], t_part[:, :H]], axis=1),
            jnp.concatenate([a_part[:, H:], t_part[:, H:]], axis=1),
        ])

    ws = halves(A1, T1)
    wd = halves(A2, T2)
    wa = halves(Ae, Te)

    wep = jnp.zeros((V_IN, D), jnp.float32).at[:, :EF].set(W_e)
    src_pair, dst_pair, xe = _node_tables(x, ws, wd, wep)
    edg_pair, ee, amx, amn = _edge_tables(e, wa, W_ee)

    smax = jnp.concatenate([src_pair[0, :, :H].max(0), src_pair[1, :, :H].max(0)])
    smin = jnp.concatenate([src_pair[0, :, :H].min(0), src_pair[1, :, :H].min(0)])
    dmax = jnp.concatenate([dst_pair[0, :, :H].max(0), dst_pair[1, :, :H].max(0)])
    dmin = jnp.concatenate([dst_pair[0, :, :H].min(0), dst_pair[1, :, :H].min(0)])
    emax = amx.max(axis=(0, 1))
    emin = amn.min(axis=(0, 1))
    hi = smax + dmax + emax
    lo = smin + dmin + emin
    mvec = jnp.maximum(hi, jnp.maximum(prelu_w * hi, prelu_w * lo))
    mvec = mvec.astype(jnp.float32)

    src_tab = src_pair.reshape(NC * N_NODES, D)
    dst_tab = dst_pair.reshape(NC * N_NODES, D)
    edg_tab = edg_pair.reshape(NC * N_EDGES, D)
    pwv = jnp.full((16,), prelu_w, jnp.float32)
    srcg = jnp.concatenate([src, src + N_NODES])
    dstg = jnp.concatenate([dst, dst + N_NODES])

    out_x, out_e = _sc_pass(src_tab, dst_tab, edg_tab, xe, ee,
                            srcg, dstg, dst,
                            mvec, b_T.astype(jnp.float32), pwv)

    new_x = jnp.concatenate([out_x[:N_NODES], out_x[NPAD:NPAD + N_NODES]],
                            axis=1)
    return (new_x, out_e)


# speedup vs baseline: 2.2057x; 1.2767x over previous
import functools

import jax
import jax.numpy as jnp
from jax import lax
from jax.experimental import pallas as pl
from jax.experimental.pallas import tpu as pltpu
from jax.experimental.pallas import tpu_sc as plsc

N_NODES = 10000
N_EDGES = 320000
V_IN = 128
D = 128
EF = 16
H = 64
NC = 2
NS = 16
EB = 40
EDGES_PER_TILE = N_EDGES // NS
NBATCH = EDGES_PER_TILE // EB
NPAD = 10240
NODES_PER_TILE = NPAD // NS
EPI_CHUNK = 64
NODE_BLK = 400
EDGE_BLK = 3200


def _node_tables_body(x_ref, ws_ref, wd_ref, we_ref, src_ref, dst_ref, xe_ref):
    xb = x_ref[...]
    src_ref[0] = jnp.dot(xb, ws_ref[0], preferred_element_type=jnp.float32)
    src_ref[1] = jnp.dot(xb, ws_ref[1], preferred_element_type=jnp.float32)
    dst_ref[0] = jnp.dot(xb, wd_ref[0], preferred_element_type=jnp.float32)
    dst_ref[1] = jnp.dot(xb, wd_ref[1], preferred_element_type=jnp.float32)
    xe_ref[...] = jnp.dot(xb, we_ref[...], preferred_element_type=jnp.float32)


def _node_tables(x, ws, wd, we):
    nblk = N_NODES // NODE_BLK
    return pl.pallas_call(
        _node_tables_body,
        grid=(nblk,),
        in_specs=[
            pl.BlockSpec((NODE_BLK, V_IN), lambda i: (i, 0)),
            pl.BlockSpec((NC, V_IN, D), lambda i: (0, 0, 0)),
            pl.BlockSpec((NC, V_IN, D), lambda i: (0, 0, 0)),
            pl.BlockSpec((V_IN, D), lambda i: (0, 0)),
        ],
        out_specs=[
            pl.BlockSpec((NC, NODE_BLK, D), lambda i: (0, i, 0)),
            pl.BlockSpec((NC, NODE_BLK, D), lambda i: (0, i, 0)),
            pl.BlockSpec((NODE_BLK, D), lambda i: (i, 0)),
        ],
        out_shape=[
            jax.ShapeDtypeStruct((NC, N_NODES, D), jnp.float32),
            jax.ShapeDtypeStruct((NC, N_NODES, D), jnp.float32),
            jax.ShapeDtypeStruct((N_NODES, D), jnp.float32),
        ],
    )(x, ws, wd, we)


def _edge_tables_body(e_ref, wa_ref, wee_ref, edg_ref, ee_ref, mx_ref, mn_ref):
    eb = e_ref[...]
    o0 = jnp.dot(eb, wa_ref[0], preferred_element_type=jnp.float32)
    o1 = jnp.dot(eb, wa_ref[1], preferred_element_type=jnp.float32)
    edg_ref[0] = o0
    edg_ref[1] = o1
    ee_ref[...] = jnp.dot(eb, wee_ref[...], preferred_element_type=jnp.float32)
    acat = jnp.concatenate([o0[:, :H], o1[:, :H]], axis=1)
    mx_ref[0] = jnp.broadcast_to(jnp.max(acat, axis=0, keepdims=True), (8, D))
    mn_ref[0] = jnp.broadcast_to(jnp.min(acat, axis=0, keepdims=True), (8, D))


def _edge_tables(e, wa, wee):
    nblk = N_EDGES // EDGE_BLK
    return pl.pallas_call(
        _edge_tables_body,
        grid=(nblk,),
        in_specs=[
            pl.BlockSpec((EDGE_BLK, EF), lambda i: (i, 0)),
            pl.BlockSpec((NC, EF, D), lambda i: (0, 0, 0)),
            pl.BlockSpec((EF, EF), lambda i: (0, 0)),
        ],
        out_specs=[
            pl.BlockSpec((NC, EDGE_BLK, D), lambda i: (0, i, 0)),
            pl.BlockSpec((EDGE_BLK, EF), lambda i: (i, 0)),
            pl.BlockSpec((1, 8, D), lambda i: (i, 0, 0)),
            pl.BlockSpec((1, 8, D), lambda i: (i, 0, 0)),
        ],
        out_shape=[
            jax.ShapeDtypeStruct((NC, N_EDGES, D), jnp.float32),
            jax.ShapeDtypeStruct((N_EDGES, EF), jnp.float32),
            jax.ShapeDtypeStruct((nblk, 8, D), jnp.float32),
            jax.ShapeDtypeStruct((nblk, 8, D), jnp.float32),
        ],
    )(e, wa, wee)


def _sc_body(src_tab, dst_tab, edg_tab, xe_tab, ee_tab,
             srcg_idx, dstg_idx, dst_idx,
             m_hbm, bt_hbm, pw_hbm,
             out_x, out_e,
             s_acc,
             srcv, dstva, dstv,
             srcrows, dstrows, edgrows, scat,
             eerows, ebo,
             mvec, btvec, pwvec,
             sem1, sem2):
    ci = lax.axis_index("c")
    si = lax.axis_index("s")
    mbase = ci * H

    pltpu.sync_copy(m_hbm, mvec)
    pltpu.sync_copy(bt_hbm, btvec)
    pltpu.sync_copy(pw_hbm, pwvec)
    pwv = pwvec[...]
    zero16 = jnp.zeros((16,), jnp.float32)

    @pl.loop(0, EB * 8)
    def _zbody(i):
        r = lax.shift_right_logical(i, 3)
        co = jnp.bitwise_and(i, 7) * 16
        scat[r, pl.ds(co, 16)] = zero16

    for k in range(NODES_PER_TILE // EB):
        base = si * NODES_PER_TILE + k * EB
        pltpu.sync_copy(scat, s_acc.at[pl.ds(base, EB)])
    plsc.subcore_barrier()

    ebase = si * EDGES_PER_TILE
    idx_off = ci * N_EDGES

    @pl.loop(0, NBATCH)
    def _batch(nb):
        start = ebase + nb * EB
        pltpu.sync_copy(srcg_idx.at[pl.ds(idx_off + start, EB)], srcv)
        pltpu.sync_copy(dstg_idx.at[pl.ds(idx_off + start, EB)], dstva)
        pltpu.sync_copy(dst_idx.at[pl.ds(start, EB)], dstv)

        cp1 = pltpu.async_copy(src_tab.at[srcv], srcrows, sem1)
        cp2 = pltpu.async_copy(dst_tab.at[dstva], dstrows, sem2)
        pltpu.sync_copy(edg_tab.at[pl.ds(idx_off + start, EB)], edgrows)
        cp1.wait()
        cp2.wait()

        mvs = [mvec[pl.ds(mbase + h * 16, 16)] for h in range(4)]

        @pl.loop(0, EB)
        def _cbody(b):
            for h in range(4):
                co = h * 16
                a1 = srcrows[b, pl.ds(co, 16)]
                a2 = dstrows[b, pl.ds(co, 16)]
                ae = edgrows[b, pl.ds(co, 16)]
                lin = a1 + a2 + ae
                logit = jnp.where(lin >= 0.0, lin, pwv * lin)
                ex = jnp.exp(logit - mvs[h])
                t1 = srcrows[b, pl.ds(co + H, 16)]
                t2 = dstrows[b, pl.ds(co + H, 16)]
                te = edgrows[b, pl.ds(co + H, 16)]
                scat[b, pl.ds(co, 16)] = ex
                scat[b, pl.ds(co + H, 16)] = ex * (t1 + t2 + te)

        pltpu.sync_copy(scat, s_acc.at[dstv], add=True)

    wid = si * NC + ci
    nbase = wid * (N_EDGES // (NC * NS))

    @pl.loop(0, N_EDGES // (NC * NS * EB))
    def _nebatch(nb):
        start = nbase + nb * EB
        pltpu.sync_copy(dst_idx.at[pl.ds(start, EB)], dstv)
        pltpu.sync_copy(srcg_idx.at[pl.ds(start, EB)], srcv)
        cp3 = pltpu.async_copy(xe_tab.at[srcv], srcrows, sem1)
        cp4 = pltpu.async_copy(xe_tab.at[dstv], dstrows, sem2)
        pltpu.sync_copy(ee_tab.at[pl.ds(start, EB)], eerows)
        cp3.wait()
        cp4.wait()

        @pl.loop(0, EB)
        def _nbody(b):
            eerows[b, :] = (srcrows[b, pl.ds(0, EF)] +
                            dstrows[b, pl.ds(0, EF)] + eerows[b, :])

        pltpu.sync_copy(eerows, out_e.at[pl.ds(start, EB)])

    plsc.subcore_barrier()

    eps = jnp.full((16,), 1e-16, jnp.float32)
    for k in range(NODES_PER_TILE // EB):
        base = si * NODES_PER_TILE + k * EB
        pltpu.sync_copy(s_acc.at[pl.ds(base, EB)], srcrows)

        bts = [btvec[pl.ds(mbase + h * 16, 16)] for h in range(4)]

        @pl.loop(0, EB)
        def _ebody(r):
            for h in range(4):
                co = h * 16
                s0 = srcrows[r, pl.ds(co, 16)]
                s1 = srcrows[r, pl.ds(co + H, 16)]
                ebo[r, pl.ds(co, 16)] = s1 / (s0 + eps) + bts[h]

        pltpu.sync_copy(ebo, out_x.at[pl.ds(ci * NPAD + base, EB)])


_sc_pass = functools.partial(
    pl.kernel,
    out_type=[
        jax.ShapeDtypeStruct((NC * NPAD, H), jnp.float32),
        jax.ShapeDtypeStruct((N_EDGES, EF), jnp.float32),
    ],
    mesh=plsc.VectorSubcoreMesh(
        core_axis_name="c", subcore_axis_name="s", num_cores=NC,
        num_subcores=NS),
    scratch_types=[
        pltpu.VMEM_SHARED((NPAD, D), jnp.float32),
        pltpu.VMEM((EB,), jnp.int32),
        pltpu.VMEM((EB,), jnp.int32),
        pltpu.VMEM((EB,), jnp.int32),
        pltpu.VMEM((EB, D), jnp.float32),
        pltpu.VMEM((EB, D), jnp.float32),
        pltpu.VMEM((EB, D), jnp.float32),
        pltpu.VMEM((EB, D), jnp.float32),
        pltpu.VMEM((EB, EF), jnp.float32),
        pltpu.VMEM((EB, H), jnp.float32),
        pltpu.VMEM((D,), jnp.float32),
        pltpu.VMEM((D,), jnp.float32),
        pltpu.VMEM((16,), jnp.float32),
        pltpu.SemaphoreType.DMA,
        pltpu.SemaphoreType.DMA,
    ],
)(_sc_body)


def kernel(x, edge_index, edge_attr, W_a, W_T, b_T, W_e, W_ee, prelu_w):
    x = x.astype(jnp.float32)
    e = edge_attr.astype(jnp.float32)
    src = edge_index[0].astype(jnp.int32)
    dst = edge_index[1].astype(jnp.int32)

    A2, Ae, A1 = W_a[:V_IN], W_a[V_IN:V_IN + EF], W_a[V_IN + EF:]
    T2, Te, T1 = W_T[:V_IN], W_T[V_IN:V_IN + EF], W_T[V_IN + EF:]

    def halves(a_part, t_part):
        return jnp.stack([
            jnp.concatenate([a_part[:, :H], t_part[:, :H]], axis=1),
            jnp.concatenate([a_part[:, H:], t_part[:, H:]], axis=1),
        ])

    ws = halves(A1, T1)
    wd = halves(A2, T2)
    wa = halves(Ae, Te)

    wep = jnp.zeros((V_IN, D), jnp.float32).at[:, :EF].set(W_e)
    src_pair, dst_pair, xe = _node_tables(x, ws, wd, wep)
    edg_pair, ee, amx, amn = _edge_tables(e, wa, W_ee)

    smax = jnp.concatenate([src_pair[0, :, :H].max(0), src_pair[1, :, :H].max(0)])
    smin = jnp.concatenate([src_pair[0, :, :H].min(0), src_pair[1, :, :H].min(0)])
    dmax = jnp.concatenate([dst_pair[0, :, :H].max(0), dst_pair[1, :, :H].max(0)])
    dmin = jnp.concatenate([dst_pair[0, :, :H].min(0), dst_pair[1, :, :H].min(0)])
    emax = amx.max(axis=(0, 1))
    emin = amn.min(axis=(0, 1))
    hi = smax + dmax + emax
    lo = smin + dmin + emin
    mvec = jnp.maximum(hi, jnp.maximum(prelu_w * hi, prelu_w * lo))
    mvec = mvec.astype(jnp.float32)

    src_tab = src_pair.reshape(NC * N_NODES, D)
    dst_tab = dst_pair.reshape(NC * N_NODES, D)
    edg_tab = edg_pair.reshape(NC * N_EDGES, D)
    pwv = jnp.full((16,), prelu_w, jnp.float32)
    srcg = jnp.concatenate([src, src + N_NODES])
    dstg = jnp.concatenate([dst, dst + N_NODES])

    out_x, out_e = _sc_pass(src_tab, dst_tab, edg_tab, xe, ee,
                            srcg, dstg, dst,
                            mvec, b_T.astype(jnp.float32), pwv)

    new_x = jnp.concatenate([out_x[:N_NODES], out_x[NPAD:NPAD + N_NODES]],
                            axis=1)
    return (new_x, out_e)


# speedup vs baseline: 2.6324x; 1.1935x over previous
import functools

import jax
import jax.numpy as jnp
from jax import lax
from jax.experimental import pallas as pl
from jax.experimental.pallas import tpu as pltpu
from jax.experimental.pallas import tpu_sc as plsc

N_NODES = 10000
N_EDGES = 320000
V_IN = 128
D = 128
EF = 16
H = 64
NC = 2
NS = 16
EB = 40
EDGES_PER_TILE = N_EDGES // NS
NBATCH = EDGES_PER_TILE // EB
NPAD = 10240
NODES_PER_TILE = NPAD // NS
EPI_CHUNK = 64
NODE_BLK = 400
EDGE_BLK = 3200


def _node_tables_body(x_ref, ws_ref, wd_ref, we_ref, src_ref, dst_ref, xe_ref):
    xb = x_ref[...]
    src_ref[0] = jnp.dot(xb, ws_ref[0], preferred_element_type=jnp.float32)
    src_ref[1] = jnp.dot(xb, ws_ref[1], preferred_element_type=jnp.float32)
    dst_ref[0] = jnp.dot(xb, wd_ref[0], preferred_element_type=jnp.float32)
    dst_ref[1] = jnp.dot(xb, wd_ref[1], preferred_element_type=jnp.float32)
    xe_ref[...] = jnp.dot(xb, we_ref[...], preferred_element_type=jnp.float32)


def _node_tables(x, ws, wd, we):
    nblk = N_NODES // NODE_BLK
    return pl.pallas_call(
        _node_tables_body,
        grid=(nblk,),
        in_specs=[
            pl.BlockSpec((NODE_BLK, V_IN), lambda i: (i, 0)),
            pl.BlockSpec((NC, V_IN, D), lambda i: (0, 0, 0)),
            pl.BlockSpec((NC, V_IN, D), lambda i: (0, 0, 0)),
            pl.BlockSpec((V_IN, D), lambda i: (0, 0)),
        ],
        out_specs=[
            pl.BlockSpec((NC, NODE_BLK, D), lambda i: (0, i, 0)),
            pl.BlockSpec((NC, NODE_BLK, D), lambda i: (0, i, 0)),
            pl.BlockSpec((NODE_BLK, D), lambda i: (i, 0)),
        ],
        out_shape=[
            jax.ShapeDtypeStruct((NC, N_NODES, D), jnp.float32),
            jax.ShapeDtypeStruct((NC, N_NODES, D), jnp.float32),
            jax.ShapeDtypeStruct((N_NODES, D), jnp.float32),
        ],
    )(x, ws, wd, we)


def _edge_tables_body(e_ref, wa_ref, wee_ref, edg_ref, ee_ref, mx_ref, mn_ref):
    eb = e_ref[...]
    o0 = jnp.dot(eb, wa_ref[0], preferred_element_type=jnp.float32)
    o1 = jnp.dot(eb, wa_ref[1], preferred_element_type=jnp.float32)
    edg_ref[0] = o0
    edg_ref[1] = o1
    ee_ref[...] = jnp.dot(eb, wee_ref[...], preferred_element_type=jnp.float32)
    acat = jnp.concatenate([o0[:, :H], o1[:, :H]], axis=1)
    mx_ref[0] = jnp.broadcast_to(jnp.max(acat, axis=0, keepdims=True), (8, D))
    mn_ref[0] = jnp.broadcast_to(jnp.min(acat, axis=0, keepdims=True), (8, D))


def _edge_tables(e, wa, wee):
    nblk = N_EDGES // EDGE_BLK
    return pl.pallas_call(
        _edge_tables_body,
        grid=(nblk,),
        in_specs=[
            pl.BlockSpec((EDGE_BLK, EF), lambda i: (i, 0)),
            pl.BlockSpec((NC, EF, D), lambda i: (0, 0, 0)),
            pl.BlockSpec((EF, EF), lambda i: (0, 0)),
        ],
        out_specs=[
            pl.BlockSpec((NC, EDGE_BLK, D), lambda i: (0, i, 0)),
            pl.BlockSpec((EDGE_BLK, EF), lambda i: (i, 0)),
            pl.BlockSpec((1, 8, D), lambda i: (i, 0, 0)),
            pl.BlockSpec((1, 8, D), lambda i: (i, 0, 0)),
        ],
        out_shape=[
            jax.ShapeDtypeStruct((NC, N_EDGES, D), jnp.float32),
            jax.ShapeDtypeStruct((N_EDGES, EF), jnp.float32),
            jax.ShapeDtypeStruct((nblk, 8, D), jnp.float32),
            jax.ShapeDtypeStruct((nblk, 8, D), jnp.float32),
        ],
    )(e, wa, wee)


def _sc_body(src_tab, dst_tab, edg_tab, xe_tab, ee_tab,
             srcg_idx, dstg_idx, dst_idx,
             m_hbm, bt_hbm, pw_hbm,
             out_x, out_e,
             s_acc,
             srcv2, dstva2, dstv2, dstsA, dstsB,
             srcrowsA, dstrowsA, edgrowsA,
             srcrowsB, dstrowsB, edgrowsB,
             scat, eerows, ebo,
             mvec, btvec, pwvec,
             semA, semB):
    ci = lax.axis_index("c")
    si = lax.axis_index("s")
    mbase = ci * H

    pltpu.sync_copy(m_hbm, mvec)
    pltpu.sync_copy(bt_hbm, btvec)
    pltpu.sync_copy(pw_hbm, pwvec)
    pwv = pwvec[...]
    zero16 = jnp.zeros((16,), jnp.float32)

    @pl.loop(0, EB * 8)
    def _zbody(i):
        r = lax.shift_right_logical(i, 3)
        co = jnp.bitwise_and(i, 7) * 16
        scat[r, pl.ds(co, 16)] = zero16

    for k in range(NODES_PER_TILE // EB):
        base = si * NODES_PER_TILE + k * EB
        pltpu.sync_copy(scat, s_acc.at[pl.ds(base, EB)])
    plsc.subcore_barrier()

    ebase = si * EDGES_PER_TILE
    idx_off = ci * N_EDGES
    mvs = [mvec[pl.ds(mbase + h * 16, 16)] for h in range(4)]

    def _copy40(dst_ref, src_ref, off):
        for c in (0, 16, 24):
            dst_ref[pl.ds(c, 16)] = src_ref[pl.ds(off + c, 16)]

    def _softmax_batch(rows_s, rows_d, rows_e, dsts):
        @pl.loop(0, EB)
        def _cbody(b):
            for h in range(4):
                co = h * 16
                a1 = rows_s[b, pl.ds(co, 16)]
                a2 = rows_d[b, pl.ds(co, 16)]
                ae = rows_e[b, pl.ds(co, 16)]
                lin = a1 + a2 + ae
                logit = jnp.where(lin >= 0.0, lin, pwv * lin)
                ex = jnp.exp(logit - mvs[h])
                t1 = rows_s[b, pl.ds(co + H, 16)]
                t2 = rows_d[b, pl.ds(co + H, 16)]
                te = rows_e[b, pl.ds(co + H, 16)]
                scat[b, pl.ds(co, 16)] = ex
                scat[b, pl.ds(co + H, 16)] = ex * (t1 + t2 + te)

        pltpu.sync_copy(scat, s_acc.at[dsts], add=True)

    @pl.loop(0, NBATCH // 2)
    def _pair(g):
        start = ebase + g * (2 * EB)
        pltpu.sync_copy(srcg_idx.at[pl.ds(idx_off + start, 2 * EB)], srcv2)
        pltpu.sync_copy(dstg_idx.at[pl.ds(idx_off + start, 2 * EB)], dstva2)
        pltpu.sync_copy(dst_idx.at[pl.ds(start, 2 * EB)], dstv2)
        _copy40(dstsA, dstv2, 0)
        _copy40(dstsB, dstv2, EB)

        cpA1 = pltpu.async_copy(src_tab.at[srcv2.at[pl.ds(0, EB)]],
                                srcrowsA, semA)
        cpA2 = pltpu.async_copy(dst_tab.at[dstva2.at[pl.ds(0, EB)]],
                                dstrowsA, semA)
        cpA3 = pltpu.async_copy(edg_tab.at[pl.ds(idx_off + start, EB)],
                                edgrowsA, semA)
        cpB1 = pltpu.async_copy(src_tab.at[srcv2.at[pl.ds(EB, EB)]],
                                srcrowsB, semB)
        cpB2 = pltpu.async_copy(dst_tab.at[dstva2.at[pl.ds(EB, EB)]],
                                dstrowsB, semB)
        cpB3 = pltpu.async_copy(edg_tab.at[pl.ds(idx_off + start + EB, EB)],
                                edgrowsB, semB)
        cpA1.wait()
        cpA2.wait()
        cpA3.wait()
        _softmax_batch(srcrowsA, dstrowsA, edgrowsA, dstsA)
        cpB1.wait()
        cpB2.wait()
        cpB3.wait()
        _softmax_batch(srcrowsB, dstrowsB, edgrowsB, dstsB)

    wid = si * NC + ci
    nbase = wid * (N_EDGES // (NC * NS))

    def _ne_batch(rows_s, rows_d, start):
        pltpu.sync_copy(ee_tab.at[pl.ds(start, EB)], eerows)

        @pl.loop(0, EB)
        def _nbody(b):
            eerows[b, :] = (rows_s[b, pl.ds(0, EF)] +
                            rows_d[b, pl.ds(0, EF)] + eerows[b, :])

        pltpu.sync_copy(eerows, out_e.at[pl.ds(start, EB)])

    @pl.loop(0, N_EDGES // (NC * NS * EB * 2))
    def _nepair(g):
        start = nbase + g * (2 * EB)
        pltpu.sync_copy(srcg_idx.at[pl.ds(start, 2 * EB)], srcv2)
        pltpu.sync_copy(dst_idx.at[pl.ds(start, 2 * EB)], dstv2)
        cpA1 = pltpu.async_copy(xe_tab.at[srcv2.at[pl.ds(0, EB)]],
                                srcrowsA, semA)
        cpA2 = pltpu.async_copy(xe_tab.at[dstv2.at[pl.ds(0, EB)]],
                                dstrowsA, semA)
        cpB1 = pltpu.async_copy(xe_tab.at[srcv2.at[pl.ds(EB, EB)]],
                                srcrowsB, semB)
        cpB2 = pltpu.async_copy(xe_tab.at[dstv2.at[pl.ds(EB, EB)]],
                                dstrowsB, semB)
        cpA1.wait()
        cpA2.wait()
        _ne_batch(srcrowsA, dstrowsA, start)
        cpB1.wait()
        cpB2.wait()
        _ne_batch(srcrowsB, dstrowsB, start + EB)

    plsc.subcore_barrier()

    eps = jnp.full((16,), 1e-16, jnp.float32)
    bts = [btvec[pl.ds(mbase + h * 16, 16)] for h in range(4)]
    for k in range(NODES_PER_TILE // EB):
        base = si * NODES_PER_TILE + k * EB
        pltpu.sync_copy(s_acc.at[pl.ds(base, EB)], srcrowsA)

        @pl.loop(0, EB)
        def _ebody(r):
            for h in range(4):
                co = h * 16
                s0 = srcrowsA[r, pl.ds(co, 16)]
                s1 = srcrowsA[r, pl.ds(co + H, 16)]
                ebo[r, pl.ds(co, 16)] = s1 / (s0 + eps) + bts[h]

        pltpu.sync_copy(ebo, out_x.at[pl.ds(ci * NPAD + base, EB)])


_sc_pass = functools.partial(
    pl.kernel,
    out_type=[
        jax.ShapeDtypeStruct((NC * NPAD, H), jnp.float32),
        jax.ShapeDtypeStruct((N_EDGES, EF), jnp.float32),
    ],
    mesh=plsc.VectorSubcoreMesh(
        core_axis_name="c", subcore_axis_name="s", num_cores=NC,
        num_subcores=NS),
    scratch_types=[
        pltpu.VMEM_SHARED((NPAD, D), jnp.float32),
        pltpu.VMEM((2 * EB,), jnp.int32),
        pltpu.VMEM((2 * EB,), jnp.int32),
        pltpu.VMEM((2 * EB,), jnp.int32),
        pltpu.VMEM((EB,), jnp.int32),
        pltpu.VMEM((EB,), jnp.int32),
        pltpu.VMEM((EB, D), jnp.float32),
        pltpu.VMEM((EB, D), jnp.float32),
        pltpu.VMEM((EB, D), jnp.float32),
        pltpu.VMEM((EB, D), jnp.float32),
        pltpu.VMEM((EB, D), jnp.float32),
        pltpu.VMEM((EB, D), jnp.float32),
        pltpu.VMEM((EB, D), jnp.float32),
        pltpu.VMEM((EB, EF), jnp.float32),
        pltpu.VMEM((EB, H), jnp.float32),
        pltpu.VMEM((D,), jnp.float32),
        pltpu.VMEM((D,), jnp.float32),
        pltpu.VMEM((16,), jnp.float32),
        pltpu.SemaphoreType.DMA,
        pltpu.SemaphoreType.DMA,
    ],
)(_sc_body)


def kernel(x, edge_index, edge_attr, W_a, W_T, b_T, W_e, W_ee, prelu_w):
    x = x.astype(jnp.float32)
    e = edge_attr.astype(jnp.float32)
    src = edge_index[0].astype(jnp.int32)
    dst = edge_index[1].astype(jnp.int32)

    A2, Ae, A1 = W_a[:V_IN], W_a[V_IN:V_IN + EF], W_a[V_IN + EF:]
    T2, Te, T1 = W_T[:V_IN], W_T[V_IN:V_IN + EF], W_T[V_IN + EF:]

    def halves(a_part, t_part):
        return jnp.stack([
            jnp.concatenate([a_part[:, :H], t_part[:, :H]], axis=1),
            jnp.concatenate([a_part[:, H:], t_part[:, H:]], axis=1),
        ])

    ws = halves(A1, T1)
    wd = halves(A2, T2)
    wa = halves(Ae, Te)

    wep = jnp.zeros((V_IN, D), jnp.float32).at[:, :EF].set(W_e)
    src_pair, dst_pair, xe = _node_tables(x, ws, wd, wep)
    edg_pair, ee, amx, amn = _edge_tables(e, wa, W_ee)

    smax = jnp.concatenate([src_pair[0, :, :H].max(0), src_pair[1, :, :H].max(0)])
    smin = jnp.concatenate([src_pair[0, :, :H].min(0), src_pair[1, :, :H].min(0)])
    dmax = jnp.concatenate([dst_pair[0, :, :H].max(0), dst_pair[1, :, :H].max(0)])
    dmin = jnp.concatenate([dst_pair[0, :, :H].min(0), dst_pair[1, :, :H].min(0)])
    emax = amx.max(axis=(0, 1))
    emin = amn.min(axis=(0, 1))
    hi = smax + dmax + emax
    lo = smin + dmin + emin
    mvec = jnp.maximum(hi, jnp.maximum(prelu_w * hi, prelu_w * lo))
    mvec = mvec.astype(jnp.float32)

    src_tab = src_pair.reshape(NC * N_NODES, D)
    dst_tab = dst_pair.reshape(NC * N_NODES, D)
    edg_tab = edg_pair.reshape(NC * N_EDGES, D)
    pwv = jnp.full((16,), prelu_w, jnp.float32)
    srcg = jnp.concatenate([src, src + N_NODES])
    dstg = jnp.concatenate([dst, dst + N_NODES])

    out_x, out_e = _sc_pass(src_tab, dst_tab, edg_tab, xe, ee,
                            srcg, dstg, dst,
                            mvec, b_T.astype(jnp.float32), pwv)

    new_x = jnp.concatenate([out_x[:N_NODES], out_x[NPAD:NPAD + N_NODES]],
                            axis=1)
    return (new_x, out_e)
